# baseline (speedup 1.0000x reference)
import functools

import jax
import jax.numpy as jnp
from jax import lax
from jax.experimental import pallas as pl
from jax.experimental.pallas import tpu as pltpu
from jax.experimental.pallas import tpu_sc as plsc

NC = 2
NS = 16
NW = NC * NS

LANES = 16
GDEPTH = 4
ODEPTH = 2


@functools.lru_cache(maxsize=None)
def _emb_call(n_b: int, n_l: int, d: int):
    assert d == 2 * LANES
    b_per_w = n_b // NW
    assert b_per_w * NW == n_b and b_per_w == 128
    assert n_l % GDEPTH == 0 and n_l % ODEPTH == 0

    mesh = plsc.VectorSubcoreMesh(
        core_axis_name="c", subcore_axis_name="s",
        num_cores=NC, num_subcores=NS)

    @functools.partial(
        pl.kernel,
        out_type=jax.ShapeDtypeStruct((n_l, d, n_b), jnp.float32),
        mesh=mesh,
        scratch_types=[
            pltpu.VMEM((n_l, b_per_w), jnp.int32),
            [pltpu.VMEM((b_per_w, d), jnp.float32)] * GDEPTH,
            [pltpu.VMEM((d, b_per_w), jnp.float32)] * ODEPTH,
            pltpu.VMEM((n_l, d), jnp.float32),
            [pltpu.SemaphoreType.DMA] * GDEPTH,
            [pltpu.SemaphoreType.DMA] * ODEPTH,
        ],
        compiler_params=pltpu.CompilerParams(
            use_tc_tiling_on_sc=False, needs_layout_passes=False),
    )
    def run(idx_hbm, table_hbm, pos_hbm, out_hbm,
            idx_v, rows_v, outt_v, pos_v, gsem, osem):
        wid = lax.axis_index("s") * NC + lax.axis_index("c")
        b0 = pl.multiple_of(wid * b_per_w, b_per_w)
        pltpu.sync_copy(pos_hbm, pos_v)
        pltpu.sync_copy(idx_hbm.at[:, pl.ds(b0, b_per_w)], idx_v)

        def fire(l, k):
            pltpu.async_copy(table_hbm.at[idx_v.at[l]], rows_v[k], gsem[k])

        for k in range(GDEPTH):
            fire(k, k)

        iota = lax.iota(jnp.int32, LANES)
        row0 = iota
        row1 = iota + LANES

        def handle(l, k, o):
            pltpu.make_async_copy(
                table_hbm.at[idx_v.at[l]], rows_v[k], gsem[k]).wait()

            p0 = pos_v[l, pl.ds(0, LANES)]
            p1 = pos_v[l, pl.ds(LANES, LANES)]
            src = rows_v[k]
            dst = outt_v[o]

            @pl.when(l >= ODEPTH)
            def _():
                pltpu.make_async_copy(
                    dst, out_hbm.at[l - ODEPTH, :, pl.ds(b0, b_per_w)],
                    osem[o]).wait()

            def tok_body(b, carry):
                col = jnp.full((LANES,), b, jnp.int32)
                v0 = src[b, pl.ds(0, LANES)] + p0
                v1 = src[b, pl.ds(LANES, LANES)] + p1
                plsc.store_scatter(dst, [row0, col], v0)
                plsc.store_scatter(dst, [row1, col], v1)
                return carry

            lax.fori_loop(0, b_per_w, tok_body, 0)

            @pl.when(l < n_l - GDEPTH)
            def _():
                fire(l + GDEPTH, k)

            pltpu.async_copy(dst, out_hbm.at[l, :, pl.ds(b0, b_per_w)],
                             osem[o])

        def step(l4, carry):
            for j in range(GDEPTH):
                l = l4 * GDEPTH + j
                handle(l, j, 0 if j % ODEPTH == 0 else 1)
            return carry

        lax.fori_loop(0, n_l // GDEPTH, step, 0)

        for o in range(ODEPTH):
            pltpu.make_async_copy(
                outt_v[o], out_hbm.at[0, :, pl.ds(b0, b_per_w)],
                osem[o]).wait()

    return run


def kernel(input, token_table, pos_table):
    b, l = input.shape
    v, d = token_table.shape
    idx_t = input.T.astype(jnp.int32)
    out = _emb_call(b, l, d)(idx_t, token_table, pos_table.astype(jnp.float32))
    return jnp.transpose(out, (2, 0, 1))
